# speedup vs baseline: 1.3903x; 1.3903x over previous
import jax
import jax.numpy as jnp
from jax.experimental import pallas as pl


def _body(x_ref, o_ref):
    o_ref[:, :, 0, :, :] = x_ref[...]
    o_ref[:, :, 1, :, :] = jnp.zeros_like(x_ref)


def kernel(x):
    B, C, H, W = x.shape
    CBLK = 64
    grid = (B, C // CBLK)
    out5 = pl.pallas_call(
        _body,
        grid=grid,
        in_specs=[pl.BlockSpec((1, CBLK, H, W), lambda b, c: (b, c, 0, 0))],
        out_specs=pl.BlockSpec((1, CBLK, 2, H, W), lambda b, c: (b, c, 0, 0, 0)),
        out_shape=jax.ShapeDtypeStruct((B, C, 2, H, W), x.dtype),
    )(x)
    return out5.reshape(B, 2 * C, H, W)
